# baseline (speedup 1.0000x reference)
import functools

import jax
import jax.numpy as jnp
import numpy as np
from jax.experimental import pallas as pl
from jax.experimental.pallas import tpu as pltpu
from jax.experimental.shard_map import shard_map
from jax.sharding import Mesh, PartitionSpec as P

_LANE = 128


def _rowdot_kernel(b_ref, x_ref, w_ref, o_ref):
    z = x_ref[...] * w_ref[...]
    o_ref[...] = jnp.sum(z, axis=2) + b_ref[0, 0]


def _pick_block(n, candidates):
    for c in candidates:
        if n % c == 0:
            return c
    return 1


def _rowdot_packed(b11, x3, w3):
    s_total, _, F = x3.shape
    s_blk = _pick_block(s_total, (64, 32, 16, 8, 4, 2, 1))
    grid = (s_total // s_blk,)
    return pl.pallas_call(
        _rowdot_kernel,
        out_shape=jax.ShapeDtypeStruct((s_total, _LANE), x3.dtype),
        grid_spec=pl.GridSpec(
            grid=grid,
            in_specs=[
                pl.BlockSpec(memory_space=pltpu.SMEM),
                pl.BlockSpec((s_blk, _LANE, F), lambda i: (i, 0, 0)),
                pl.BlockSpec((1, 1, F), lambda i: (0, 0, 0)),
            ],
            out_specs=pl.BlockSpec((s_blk, _LANE), lambda i: (i, 0)),
        ),
        compiler_params=pltpu.CompilerParams(
            dimension_semantics=("arbitrary",),
        ),
        cost_estimate=pl.CostEstimate(
            flops=2 * s_total * _LANE * F,
            transcendentals=0,
            bytes_accessed=s_total * _LANE * F * 4 + F * 4 + s_total * _LANE * 4,
        ),
    )(b11, x3, w3)


def kernel(x, wt_padded, b_padded):
    B, F = x.shape

    n_rows = B
    pad = (-n_rows) % _LANE
    if pad:
        x = jnp.pad(x, ((0, pad), (0, 0)))
        B = x.shape[0]

    s_total = B // _LANE
    x3 = x.reshape(s_total, _LANE, F)
    w3 = wt_padded[:, :1].reshape(1, 1, F)
    b11 = b_padded[:1, :1]

    devs = jax.devices()
    n_dev = 2 if (len(devs) >= 2 and s_total % 2 == 0) else 1
    if n_dev == 2:
        mesh = Mesh(np.array(devs[:2]), ("d",))
        fn = shard_map(
            _rowdot_packed,
            mesh=mesh,
            in_specs=(P(None, None), P("d", None, None), P(None, None, None)),
            out_specs=P("d", None),
            check_rep=False,
        )
        out = fn(b11, x3, w3)
    else:
        out = _rowdot_packed(b11, x3, w3)

    return out.reshape(B, 1)[:n_rows]


# speedup vs baseline: 17.1756x; 17.1756x over previous
import jax
import jax.numpy as jnp
from jax.experimental import pallas as pl
from jax.experimental.pallas import tpu as pltpu

_LANE = 128


def _rowdot_kernel(b_ref, x_ref, w_ref, o_ref):
    z = x_ref[...] * w_ref[...]
    o_ref[...] = jnp.sum(z, axis=2) + b_ref[0, 0]


def _pick_block(n, candidates):
    for c in candidates:
        if n % c == 0:
            return c
    return 1


def kernel(x, wt_padded, b_padded):
    B, F = x.shape
    dtype = x.dtype

    n_rows = B
    pad = (-n_rows) % _LANE
    if pad:
        x = jnp.pad(x, ((0, pad), (0, 0)))
        B = x.shape[0]

    s_total = B // _LANE
    x3 = x.reshape(s_total, _LANE, F)
    w3 = wt_padded[:, :1].reshape(1, 1, F)
    b11 = b_padded[:1, :1]

    s_blk = _pick_block(s_total, (64, 32, 16, 8, 4, 2, 1))
    grid = (s_total // s_blk,)

    out = pl.pallas_call(
        _rowdot_kernel,
        out_shape=jax.ShapeDtypeStruct((s_total, _LANE), dtype),
        grid_spec=pl.GridSpec(
            grid=grid,
            in_specs=[
                pl.BlockSpec(memory_space=pltpu.SMEM),
                pl.BlockSpec((s_blk, _LANE, F), lambda i: (i, 0, 0)),
                pl.BlockSpec((1, 1, F), lambda i: (0, 0, 0)),
            ],
            out_specs=pl.BlockSpec((s_blk, _LANE), lambda i: (i, 0)),
        ),
        compiler_params=pltpu.CompilerParams(
            dimension_semantics=("arbitrary",),
        ),
        cost_estimate=pl.CostEstimate(
            flops=2 * B * F,
            transcendentals=0,
            bytes_accessed=B * F * 4 + F * 4 + B * 4,
        ),
    )(b11, x3, w3)

    return out.reshape(B, 1)[:n_rows]
